# Initial kernel scaffold; baseline (speedup 1.0000x reference)
import jax
import jax.numpy as jnp
from jax.experimental import pallas as pl


def kernel(x, edge_index, params):
    raise NotImplementedError("write your pallas kernel here")


# speedup vs baseline: 1.0687x; 1.0687x over previous
import functools

import jax
import jax.numpy as jnp
from jax.experimental import pallas as pl

HEADS = 8
HCH = 32


def _dense_body(act, x_ref, w_ref, b_ref, o_ref):
    h = jnp.dot(x_ref[...], w_ref[...], preferred_element_type=jnp.float32)
    h = h + b_ref[...]
    if act == "relu":
        h = jnp.maximum(h, 0.0)
    elif act == "sigmoid":
        h = jax.nn.sigmoid(h)
    o_ref[...] = h


def _dense(x, W, b, act=None, block_m=1024):
    M, K = x.shape
    N = W.shape[1]
    return pl.pallas_call(
        functools.partial(_dense_body, act),
        grid=(pl.cdiv(M, block_m),),
        in_specs=[
            pl.BlockSpec((block_m, K), lambda i: (i, 0)),
            pl.BlockSpec((K, N), lambda i: (0, 0)),
            pl.BlockSpec((1, N), lambda i: (0, 0)),
        ],
        out_specs=pl.BlockSpec((block_m, N), lambda i: (i, 0)),
        out_shape=jax.ShapeDtypeStruct((M, N), jnp.float32),
    )(x, W, b.reshape(1, N))


def _gcn_layer(g, src, dst, dis, lp, n):
    h = _dense(g, lp["W"], lp["b"] * 0.0)
    hp = h * dis[:, None]
    agg = jax.ops.segment_sum(hp[src], dst, num_segments=n)
    return jnp.maximum(agg * dis[:, None] + lp["b"], 0.0)


def _gat_layer(a, src, dst, lp, n):
    h = _dense(a, lp["W"], lp["b"] * 0.0).reshape(n, HEADS, HCH)
    a_s = jnp.sum(h * lp["att_s"][None], axis=-1)
    a_d = jnp.sum(h * lp["att_d"][None], axis=-1)
    alpha = jax.nn.leaky_relu(a_s[src] + a_d[dst], 0.2)
    amax = jax.ops.segment_max(alpha, dst, num_segments=n)
    amax = jnp.where(jnp.isfinite(amax), amax, 0.0)
    ex = jnp.exp(alpha - amax[dst])
    den = jax.ops.segment_sum(ex, dst, num_segments=n)
    num = jax.ops.segment_sum(h[src] * ex[:, :, None], dst, num_segments=n)
    out = num / (den[:, :, None] + 1e-16)
    return jnp.maximum(out.reshape(n, HEADS * HCH) + lp["b"], 0.0)


def _sage_layer(s, src0, dst0, inv_cnt, lp, n):
    agg = jax.ops.segment_sum(s[src0], dst0, num_segments=n) * inv_cnt[:, None]
    out = _dense(agg, lp["Wl"], lp["bl"]) + _dense(s, lp["Wr"], lp["bl"] * 0.0)
    return jnp.maximum(out, 0.0)


def kernel(x, edge_index, params):
    n = x.shape[0]
    src0, dst0 = edge_index[0], edge_index[1]
    loop = jnp.arange(n, dtype=src0.dtype)
    src = jnp.concatenate([src0, loop])
    dst = jnp.concatenate([dst0, loop])

    deg = jax.ops.segment_sum(jnp.ones_like(src, dtype=jnp.float32), dst,
                              num_segments=n)
    dis = jnp.where(deg > 0, 1.0 / jnp.sqrt(jnp.maximum(deg, 1e-12)), 0.0)
    cnt = jax.ops.segment_sum(jnp.ones_like(src0, dtype=jnp.float32), dst0,
                              num_segments=n)
    inv_cnt = 1.0 / jnp.maximum(cnt, 1.0)

    g = x
    for lp in params["gcn"]:
        g = _gcn_layer(g, src, dst, dis, lp, n)
    a = x
    for lp in params["gat"]:
        a = _gat_layer(a, src, dst, lp, n)
    s_ = x
    for lp in params["sage"]:
        s_ = _sage_layer(s_, src0, dst0, inv_cnt, lp, n)

    f = jnp.concatenate([g, a, s_], axis=1)
    f = _dense(f, params["f1W"], params["f1b"], act="relu")
    f = _dense(f, params["f2W"], params["f2b"], act="relu")
    warp = _dense(f, params["wW"], params["wb"])
    v = _dense(f, params["v1W"], params["v1b"], act="relu")
    v = _dense(v, params["v2W"], params["v2b"], act="sigmoid")
    return warp, v


# speedup vs baseline: 1.1334x; 1.0606x over previous
import functools

import jax
import jax.numpy as jnp
from jax import lax
from jax.experimental import pallas as pl
from jax.experimental.pallas import tpu as pltpu
from jax.experimental.pallas import tpu_sc as plsc

HEADS = 8
HCH = 32
NS = 16
LANES = 16
CHUNK = 128


def _dense_body(act, x_ref, w_ref, b_ref, o_ref):
    h = jnp.dot(x_ref[...], w_ref[...], preferred_element_type=jnp.float32)
    h = h + b_ref[...]
    if act == "relu":
        h = jnp.maximum(h, 0.0)
    elif act == "sigmoid":
        h = jax.nn.sigmoid(h)
    o_ref[...] = h


def _dense(x, W, b, act=None, block_m=1024):
    M, K = x.shape
    N = W.shape[1]
    return pl.pallas_call(
        functools.partial(_dense_body, act),
        grid=(pl.cdiv(M, block_m),),
        in_specs=[
            pl.BlockSpec((block_m, K), lambda i: (i, 0)),
            pl.BlockSpec((K, N), lambda i: (0, 0)),
            pl.BlockSpec((1, N), lambda i: (0, 0)),
        ],
        out_specs=pl.BlockSpec((block_m, N), lambda i: (i, 0)),
        out_shape=jax.ShapeDtypeStruct((M, N), jnp.float32),
    )(x, W, b.reshape(1, N))


def _dense_rowscale_body(x_ref, w_ref, s_ref, o_ref):
    h = jnp.dot(x_ref[...], w_ref[...], preferred_element_type=jnp.float32)
    o_ref[...] = h * s_ref[...]


def _dense_rowscale(x, W, rows, block_m=1024):
    M, K = x.shape
    N = W.shape[1]
    return pl.pallas_call(
        _dense_rowscale_body,
        grid=(pl.cdiv(M, block_m),),
        in_specs=[
            pl.BlockSpec((block_m, K), lambda i: (i, 0)),
            pl.BlockSpec((K, N), lambda i: (0, 0)),
            pl.BlockSpec((block_m, 1), lambda i: (i, 0)),
        ],
        out_specs=pl.BlockSpec((block_m, N), lambda i: (i, 0)),
        out_shape=jax.ShapeDtypeStruct((M, N), jnp.float32),
    )(x, W, rows.reshape(M, 1))


def _gcn_finish_body(agg_ref, hp_ref, dis_ref, b_ref, o_ref):
    o_ref[...] = jnp.maximum(
        (agg_ref[...] + hp_ref[...]) * dis_ref[...] + b_ref[...], 0.0)


def _gcn_finish(agg, hp, dis, b, block_m=1024):
    M, N = agg.shape
    return pl.pallas_call(
        _gcn_finish_body,
        grid=(pl.cdiv(M, block_m),),
        in_specs=[
            pl.BlockSpec((block_m, N), lambda i: (i, 0)),
            pl.BlockSpec((block_m, N), lambda i: (i, 0)),
            pl.BlockSpec((block_m, 1), lambda i: (i, 0)),
            pl.BlockSpec((1, N), lambda i: (0, 0)),
        ],
        out_specs=pl.BlockSpec((block_m, N), lambda i: (i, 0)),
        out_shape=jax.ShapeDtypeStruct((M, N), jnp.float32),
    )(agg, hp, dis.reshape(M, 1), b.reshape(1, N))


def _sage_finish_body(agg_ref, hr_ref, ic_ref, b_ref, o_ref):
    o_ref[...] = jnp.maximum(
        agg_ref[...] * ic_ref[...] + hr_ref[...] + b_ref[...], 0.0)


def _sage_finish(agg, hr, inv_cnt, b, block_m=1024):
    M, N = agg.shape
    return pl.pallas_call(
        _sage_finish_body,
        grid=(pl.cdiv(M, block_m),),
        in_specs=[
            pl.BlockSpec((block_m, N), lambda i: (i, 0)),
            pl.BlockSpec((block_m, N), lambda i: (i, 0)),
            pl.BlockSpec((block_m, 1), lambda i: (i, 0)),
            pl.BlockSpec((1, N), lambda i: (0, 0)),
        ],
        out_specs=pl.BlockSpec((block_m, N), lambda i: (i, 0)),
        out_shape=jax.ShapeDtypeStruct((M, N), jnp.float32),
    )(agg, hr, inv_cnt.reshape(M, 1), b.reshape(1, N))


def _degs_body(cnt_ref, dis_ref, ic_ref):
    c = cnt_ref[...]
    dis_ref[...] = lax.rsqrt(c + 1.0)
    ic_ref[...] = 1.0 / jnp.maximum(c, 1.0)


def _degs(cnt):
    M = cnt.shape[0]
    c2 = cnt.reshape(M // 128, 128)
    out = pl.pallas_call(
        _degs_body,
        out_shape=(jax.ShapeDtypeStruct(c2.shape, jnp.float32),
                   jax.ShapeDtypeStruct(c2.shape, jnp.float32)),
    )(c2)
    return out[0].reshape(M), out[1].reshape(M)


def _sc_mesh():
    return plsc.VectorSubcoreMesh(
        core_axis_name="c", subcore_axis_name="s", num_cores=2,
        num_subcores=NS)


def _zero_vmem(buf, nrows, ncols):
    zero = jnp.zeros((LANES,), jnp.float32)
    vpr = ncols // LANES

    def z(i, _):
        buf[i // vpr, pl.ds((i % vpr) * LANES, LANES)] = zero
        return 0

    lax.fori_loop(0, nrows * vpr, z, 0)


def _seg_sum_sc(h2, srcT2, dstT2, npad):
    nchunks = srcT2.shape[0] // NS
    rpt = npad // NS

    def body(h_ref, src_hbm, dst_hbm, out_ref,
             sidx, didx, tidx, gbuf, widx, acc,
             semi0, semi1, semd0, semd1, semg0, semg1):
        cc = lax.axis_index("c")
        ss = lax.axis_index("s")
        semi = (semi0, semi1)
        semd = (semd0, semd1)
        semg = (semg0, semg1)
        iota = lax.iota(jnp.int32, LANES)

        zero = jnp.zeros((LANES,), jnp.float32)
        def zg(i, _):
            gbuf[0, i // 8, pl.ds((i % 8) * LANES, LANES)] = zero
            return 0
        lax.fori_loop(0, CHUNK * 8, zg, 0)
        def zacc(p, _):
            pltpu.sync_copy(
                gbuf.at[0], acc.at[pl.ds(ss * rpt + p * CHUNK, CHUNK)])
            return 0
        lax.fori_loop(0, rpt // CHUNK, zacc, 0)

        def idx_dma(j, b):
            pltpu.make_async_copy(
                src_hbm.at[ss * nchunks + j], sidx.at[b], semi[b]).start()
            pltpu.make_async_copy(
                dst_hbm.at[ss * nchunks + j], didx.at[b], semd[b]).start()

        def idx_wait_transform(j, b):
            pltpu.make_async_copy(
                src_hbm.at[ss * nchunks + j], sidx.at[b], semi[b]).wait()
            pltpu.make_async_copy(
                dst_hbm.at[ss * nchunks + j], didx.at[b], semd[b]).wait()
            def tr(k, _):
                tidx[b, pl.ds(k * LANES, LANES)] = (
                    sidx[b, pl.ds(k * LANES, LANES)] * 2 + cc)
                return 0
            lax.fori_loop(0, CHUNK // LANES, tr, 0)

        def gather_start(j, b):
            pltpu.make_async_copy(
                h_ref.at[tidx.at[b]], gbuf.at[b], semg[b]).start()

        def gather_wait_scatter(j, b):
            pltpu.make_async_copy(
                h_ref.at[tidx.at[b]], gbuf.at[b], semg[b]).wait()
            pltpu.sync_copy(gbuf.at[b], acc.at[didx.at[b]], add=True)

        plsc.subcore_barrier()

        idx_dma(0, 0)
        idx_wait_transform(0, 0)
        gather_start(0, 0)
        if nchunks > 1:
            idx_dma(1, 1)

        def step(jj, _):
            j = jj * 2
            for b in range(2):
                jb = j + b
                nb = 1 - b
                @pl.when(jb + 1 < nchunks)
                def _():
                    idx_wait_transform(jb + 1, nb)
                    gather_start(jb + 1, nb)
                @pl.when(jb < nchunks)
                def _():
                    gather_wait_scatter(jb, b)
                @pl.when(jb + 2 < nchunks)
                def _():
                    idx_dma(jb + 2, b)
            return 0
        lax.fori_loop(0, (nchunks + 1) // 2, step, 0)

        plsc.subcore_barrier()

        def wb(p, _):
            def mkidx(k, _):
                widx[0, pl.ds(k * LANES, LANES)] = (
                    iota + (ss * rpt + p * CHUNK + k * LANES)) * 2 + cc
                return 0
            lax.fori_loop(0, CHUNK // LANES, mkidx, 0)
            pltpu.sync_copy(
                acc.at[pl.ds(ss * rpt + p * CHUNK, CHUNK)], gbuf.at[0])
            pltpu.sync_copy(gbuf.at[0], out_ref.at[widx.at[0]])
            return 0
        lax.fori_loop(0, rpt // CHUNK, wb, 0)

    f = pl.kernel(
        body,
        out_type=jax.ShapeDtypeStruct((2 * npad, 128), jnp.float32),
        mesh=_sc_mesh(),
        scratch_types=[
            pltpu.VMEM((2, CHUNK), jnp.int32),
            pltpu.VMEM((2, CHUNK), jnp.int32),
            pltpu.VMEM((2, CHUNK), jnp.int32),
            pltpu.VMEM((2, CHUNK, 128), jnp.float32),
            pltpu.VMEM((1, CHUNK), jnp.int32),
            pltpu.VMEM_SHARED((npad, 128), jnp.float32),
            pltpu.SemaphoreType.DMA,
            pltpu.SemaphoreType.DMA,
            pltpu.SemaphoreType.DMA,
            pltpu.SemaphoreType.DMA,
            pltpu.SemaphoreType.DMA,
            pltpu.SemaphoreType.DMA,
        ],
    )
    return f(h2, srcT2, dstT2)


def _edge_count_sc(dstT, npad):
    nchunks = dstT.shape[1]
    rpt = npad // NS

    def body(dst_hbm, out_ref, dst_v, ones_v, stage, acc1, sem0):
        cc = lax.axis_index("c")
        ss = lax.axis_index("s")

        @pl.when(cc == 0)
        def _():
            def zs(i, _):
                stage[pl.ds(i * LANES, LANES)] = jnp.zeros((LANES,), jnp.float32)
                return 0
            lax.fori_loop(0, rpt // LANES, zs, 0)
            pltpu.sync_copy(stage, acc1.at[pl.ds(ss * rpt, rpt)])

            def os(i, _):
                ones_v[pl.ds(i * LANES, LANES)] = jnp.ones((LANES,), jnp.float32)
                return 0
            lax.fori_loop(0, CHUNK // LANES, os, 0)

            pltpu.sync_copy(dst_hbm.at[ss], dst_v)
            plsc.subcore_barrier()

            def step(j, _):
                pltpu.sync_copy(ones_v, acc1.at[dst_v.at[j]], add=True)
                return 0
            lax.fori_loop(0, nchunks, step, 0)

            plsc.subcore_barrier()
            pltpu.sync_copy(acc1.at[pl.ds(ss * rpt, rpt)],
                            out_ref.at[pl.ds(ss * rpt, rpt)])

        @pl.when(cc != 0)
        def _():
            plsc.subcore_barrier()
            plsc.subcore_barrier()

    f = pl.kernel(
        body,
        out_type=jax.ShapeDtypeStruct((npad,), jnp.float32),
        mesh=_sc_mesh(),
        scratch_types=[
            pltpu.VMEM((nchunks, CHUNK), jnp.int32),
            pltpu.VMEM((CHUNK,), jnp.float32),
            pltpu.VMEM((rpt,), jnp.float32),
            pltpu.VMEM_SHARED((npad,), jnp.float32),
            pltpu.SemaphoreType.DMA,
        ],
    )
    return f(dstT)


def _gat_layer_xla(a, src, dst, lp, n):
    h = _dense(a, lp["W"], lp["b"] * 0.0).reshape(n, HEADS, HCH)
    a_s = jnp.sum(h * lp["att_s"][None], axis=-1)
    a_d = jnp.sum(h * lp["att_d"][None], axis=-1)
    alpha = jax.nn.leaky_relu(a_s[src] + a_d[dst], 0.2)
    amax = jax.ops.segment_max(alpha, dst, num_segments=n)
    amax = jnp.where(jnp.isfinite(amax), amax, 0.0)
    ex = jnp.exp(alpha - amax[dst])
    den = jax.ops.segment_sum(ex, dst, num_segments=n)
    num = jax.ops.segment_sum(h[src] * ex[:, :, None], dst, num_segments=n)
    out = num / (den[:, :, None] + 1e-16)
    return jnp.maximum(out.reshape(n, HEADS * HCH) + lp["b"], 0.0)


def kernel(x, edge_index, params):
    n = x.shape[0]
    src0, dst0 = edge_index[0], edge_index[1]
    E = src0.shape[0]

    npad = ((n + 16 + NS * 128 - 1) // (NS * 128)) * (NS * 128)
    ept = ((E + NS * CHUNK - 1) // (NS * CHUNK)) * CHUNK
    epad = ept * NS
    pad = epad - E
    pidx = jnp.arange(pad, dtype=src0.dtype)
    src_p = jnp.concatenate([src0, pidx % n])
    dst_p = jnp.concatenate([dst0, n + (pidx % 16)])
    srcT = src_p.reshape(NS, ept // CHUNK, CHUNK)
    dstT = dst_p.reshape(NS, ept // CHUNK, CHUNK)
    srcT2 = src_p.reshape(NS * (ept // CHUNK), CHUNK)
    dstT2 = dst_p.reshape(NS * (ept // CHUNK), CHUNK)

    x_pad = jnp.pad(x, ((0, npad - n), (0, 0)))

    cnt = _edge_count_sc(dstT, npad)
    dis, inv_cnt = _degs(cnt)

    def agg256(h):
        out2 = _seg_sum_sc(h.reshape(2 * npad, 128), srcT2, dstT2, npad)
        return out2.reshape(npad, 256)

    g = x_pad
    for lp in params["gcn"]:
        hp = _dense_rowscale(g, lp["W"], dis)
        agg = agg256(hp)
        g = _gcn_finish(agg, hp, dis, lp["b"])

    s_ = x_pad
    for lp in params["sage"]:
        hl = _dense(s_, lp["Wl"], lp["bl"] * 0.0)
        hr = _dense(s_, lp["Wr"], lp["bl"] * 0.0)
        agg = agg256(hl)
        s_ = _sage_finish(agg, hr, inv_cnt, lp["bl"])

    loop = jnp.arange(n, dtype=src0.dtype)
    srcl = jnp.concatenate([src0, loop])
    dstl = jnp.concatenate([dst0, loop])
    a = x
    for lp in params["gat"]:
        a = _gat_layer_xla(a, srcl, dstl, lp, n)

    f = jnp.concatenate([g[:n], a, s_[:n]], axis=1)
    f = _dense(f, params["f1W"], params["f1b"], act="relu")
    f = _dense(f, params["f2W"], params["f2b"], act="relu")
    warp = _dense(f, params["wW"], params["wb"])
    v = _dense(f, params["v1W"], params["v1b"], act="relu")
    v = _dense(v, params["v2W"], params["v2b"], act="sigmoid")
    return warp, v


# speedup vs baseline: 1.1334x; 1.0000x over previous
import functools

import jax
import jax.numpy as jnp
from jax import lax
from jax.experimental import pallas as pl
from jax.experimental.pallas import tpu as pltpu
from jax.experimental.pallas import tpu_sc as plsc

HEADS = 8
HCH = 32
NS = 16
LANES = 16
CHUNK = 128


def _dense_body(act, x_ref, w_ref, b_ref, o_ref):
    h = jnp.dot(x_ref[...], w_ref[...], preferred_element_type=jnp.float32)
    h = h + b_ref[...]
    if act == "relu":
        h = jnp.maximum(h, 0.0)
    elif act == "sigmoid":
        h = jax.nn.sigmoid(h)
    o_ref[...] = h


def _dense(x, W, b, act=None, block_m=1024):
    M, K = x.shape
    N = W.shape[1]
    return pl.pallas_call(
        functools.partial(_dense_body, act),
        grid=(pl.cdiv(M, block_m),),
        in_specs=[
            pl.BlockSpec((block_m, K), lambda i: (i, 0)),
            pl.BlockSpec((K, N), lambda i: (0, 0)),
            pl.BlockSpec((1, N), lambda i: (0, 0)),
        ],
        out_specs=pl.BlockSpec((block_m, N), lambda i: (i, 0)),
        out_shape=jax.ShapeDtypeStruct((M, N), jnp.float32),
    )(x, W, b.reshape(1, N))


def _dense_rowscale_body(x_ref, w_ref, s_ref, o_ref):
    h = jnp.dot(x_ref[...], w_ref[...], preferred_element_type=jnp.float32)
    o_ref[...] = h * s_ref[...]


def _dense_rowscale(x, W, rows, block_m=1024):
    M, K = x.shape
    N = W.shape[1]
    return pl.pallas_call(
        _dense_rowscale_body,
        grid=(pl.cdiv(M, block_m),),
        in_specs=[
            pl.BlockSpec((block_m, K), lambda i: (i, 0)),
            pl.BlockSpec((K, N), lambda i: (0, 0)),
            pl.BlockSpec((block_m, 1), lambda i: (i, 0)),
        ],
        out_specs=pl.BlockSpec((block_m, N), lambda i: (i, 0)),
        out_shape=jax.ShapeDtypeStruct((M, N), jnp.float32),
    )(x, W, rows.reshape(M, 1))


def _gcn_finish_body(agg_ref, hp_ref, dis_ref, b_ref, o_ref):
    o_ref[...] = jnp.maximum(
        (agg_ref[...] + hp_ref[...]) * dis_ref[...] + b_ref[...], 0.0)


def _gcn_finish(agg, hp, dis, b, block_m=1024):
    M, N = agg.shape
    return pl.pallas_call(
        _gcn_finish_body,
        grid=(pl.cdiv(M, block_m),),
        in_specs=[
            pl.BlockSpec((block_m, N), lambda i: (i, 0)),
            pl.BlockSpec((block_m, N), lambda i: (i, 0)),
            pl.BlockSpec((block_m, 1), lambda i: (i, 0)),
            pl.BlockSpec((1, N), lambda i: (0, 0)),
        ],
        out_specs=pl.BlockSpec((block_m, N), lambda i: (i, 0)),
        out_shape=jax.ShapeDtypeStruct((M, N), jnp.float32),
    )(agg, hp, dis.reshape(M, 1), b.reshape(1, N))


def _sage_finish_body(agg_ref, hr_ref, ic_ref, b_ref, o_ref):
    o_ref[...] = jnp.maximum(
        agg_ref[...] * ic_ref[...] + hr_ref[...] + b_ref[...], 0.0)


def _sage_finish(agg, hr, inv_cnt, b, block_m=1024):
    M, N = agg.shape
    return pl.pallas_call(
        _sage_finish_body,
        grid=(pl.cdiv(M, block_m),),
        in_specs=[
            pl.BlockSpec((block_m, N), lambda i: (i, 0)),
            pl.BlockSpec((block_m, N), lambda i: (i, 0)),
            pl.BlockSpec((block_m, 1), lambda i: (i, 0)),
            pl.BlockSpec((1, N), lambda i: (0, 0)),
        ],
        out_specs=pl.BlockSpec((block_m, N), lambda i: (i, 0)),
        out_shape=jax.ShapeDtypeStruct((M, N), jnp.float32),
    )(agg, hr, inv_cnt.reshape(M, 1), b.reshape(1, N))


def _degs_body(cnt_ref, dis_ref, ic_ref):
    c = cnt_ref[...]
    dis_ref[...] = lax.rsqrt(c + 1.0)
    ic_ref[...] = 1.0 / jnp.maximum(c, 1.0)


def _degs(cnt):
    M = cnt.shape[0]
    c2 = cnt.reshape(M // 128, 128)
    out = pl.pallas_call(
        _degs_body,
        out_shape=(jax.ShapeDtypeStruct(c2.shape, jnp.float32),
                   jax.ShapeDtypeStruct(c2.shape, jnp.float32)),
    )(c2)
    return out[0].reshape(M), out[1].reshape(M)


def _sc_mesh():
    return plsc.VectorSubcoreMesh(
        core_axis_name="c", subcore_axis_name="s", num_cores=2,
        num_subcores=NS)


def _zero_vmem(buf, nrows, ncols):
    zero = jnp.zeros((LANES,), jnp.float32)
    vpr = ncols // LANES

    def z(i, _):
        buf[i // vpr, pl.ds((i % vpr) * LANES, LANES)] = zero
        return 0

    lax.fori_loop(0, nrows * vpr, z, 0)


def _seg_sum_sc(h2, srcT2, dstT2, npad):
    nchunks = srcT2.shape[0] // NS
    rpt = npad // NS

    def body(h_ref, src_hbm, dst_hbm, out_ref,
             sidx, didx, tidx, gbuf, widx, acc,
             semi0, semi1, semd0, semd1, semg0, semg1):
        cc = lax.axis_index("c")
        ss = lax.axis_index("s")
        semi = (semi0, semi1)
        semd = (semd0, semd1)
        semg = (semg0, semg1)
        iota = lax.iota(jnp.int32, LANES)

        zero = jnp.zeros((LANES,), jnp.float32)
        def zg(i, _):
            gbuf[0, i // 8, pl.ds((i % 8) * LANES, LANES)] = zero
            return 0
        lax.fori_loop(0, CHUNK * 8, zg, 0)
        def zacc(p, _):
            pltpu.sync_copy(
                gbuf.at[0], acc.at[pl.ds(ss * rpt + p * CHUNK, CHUNK)])
            return 0
        lax.fori_loop(0, rpt // CHUNK, zacc, 0)

        def idx_dma(j, b):
            pltpu.make_async_copy(
                src_hbm.at[ss * nchunks + j], sidx.at[b], semi[b]).start()
            pltpu.make_async_copy(
                dst_hbm.at[ss * nchunks + j], didx.at[b], semd[b]).start()

        def idx_wait_transform(j, b):
            pltpu.make_async_copy(
                src_hbm.at[ss * nchunks + j], sidx.at[b], semi[b]).wait()
            pltpu.make_async_copy(
                dst_hbm.at[ss * nchunks + j], didx.at[b], semd[b]).wait()
            def tr(k, _):
                tidx[b, pl.ds(k * LANES, LANES)] = (
                    sidx[b, pl.ds(k * LANES, LANES)] * 2 + cc)
                return 0
            lax.fori_loop(0, CHUNK // LANES, tr, 0)

        def gather_start(j, b):
            pltpu.make_async_copy(
                h_ref.at[tidx.at[b]], gbuf.at[b], semg[b]).start()

        def gather_wait_scatter(j, b):
            pltpu.make_async_copy(
                h_ref.at[tidx.at[b]], gbuf.at[b], semg[b]).wait()
            pltpu.sync_copy(gbuf.at[b], acc.at[didx.at[b]], add=True)

        plsc.subcore_barrier()

        idx_dma(0, 0)
        idx_wait_transform(0, 0)
        gather_start(0, 0)
        if nchunks > 1:
            idx_dma(1, 1)

        def step(jj, _):
            j = jj * 2
            for b in range(2):
                jb = j + b
                nb = 1 - b
                @pl.when(jb + 1 < nchunks)
                def _():
                    idx_wait_transform(jb + 1, nb)
                    gather_start(jb + 1, nb)
                @pl.when(jb < nchunks)
                def _():
                    gather_wait_scatter(jb, b)
                @pl.when(jb + 2 < nchunks)
                def _():
                    idx_dma(jb + 2, b)
            return 0
        lax.fori_loop(0, (nchunks + 1) // 2, step, 0)

        plsc.subcore_barrier()

        def wb(p, _):
            def mkidx(k, _):
                widx[0, pl.ds(k * LANES, LANES)] = (
                    iota + (ss * rpt + p * CHUNK + k * LANES)) * 2 + cc
                return 0
            lax.fori_loop(0, CHUNK // LANES, mkidx, 0)
            pltpu.sync_copy(
                acc.at[pl.ds(ss * rpt + p * CHUNK, CHUNK)], gbuf.at[0])
            pltpu.sync_copy(gbuf.at[0], out_ref.at[widx.at[0]])
            return 0
        lax.fori_loop(0, rpt // CHUNK, wb, 0)

    f = pl.kernel(
        body,
        out_type=jax.ShapeDtypeStruct((2 * npad, 128), jnp.float32),
        mesh=_sc_mesh(),
        scratch_types=[
            pltpu.VMEM((2, CHUNK), jnp.int32),
            pltpu.VMEM((2, CHUNK), jnp.int32),
            pltpu.VMEM((2, CHUNK), jnp.int32),
            pltpu.VMEM((2, CHUNK, 128), jnp.float32),
            pltpu.VMEM((1, CHUNK), jnp.int32),
            pltpu.VMEM_SHARED((npad, 128), jnp.float32),
            pltpu.SemaphoreType.DMA,
            pltpu.SemaphoreType.DMA,
            pltpu.SemaphoreType.DMA,
            pltpu.SemaphoreType.DMA,
            pltpu.SemaphoreType.DMA,
            pltpu.SemaphoreType.DMA,
        ],
    )
    return f(h2, srcT2, dstT2)


def _edge_count_sc(dstT, npad):
    nchunks = dstT.shape[1]
    rpt = npad // NS

    def body(dst_hbm, out_ref, dst_v, ones_v, stage, acc1, sem0):
        cc = lax.axis_index("c")
        ss = lax.axis_index("s")

        @pl.when(cc == 0)
        def _():
            def zs(i, _):
                stage[pl.ds(i * LANES, LANES)] = jnp.zeros((LANES,), jnp.float32)
                return 0
            lax.fori_loop(0, rpt // LANES, zs, 0)
            pltpu.sync_copy(stage, acc1.at[pl.ds(ss * rpt, rpt)])

            def os(i, _):
                ones_v[pl.ds(i * LANES, LANES)] = jnp.ones((LANES,), jnp.float32)
                return 0
            lax.fori_loop(0, CHUNK // LANES, os, 0)

            pltpu.sync_copy(dst_hbm.at[ss], dst_v)
            plsc.subcore_barrier()

            def step(j, _):
                pltpu.sync_copy(ones_v, acc1.at[dst_v.at[j]], add=True)
                return 0
            lax.fori_loop(0, nchunks, step, 0)

            plsc.subcore_barrier()
            pltpu.sync_copy(acc1.at[pl.ds(ss * rpt, rpt)],
                            out_ref.at[pl.ds(ss * rpt, rpt)])

        @pl.when(cc != 0)
        def _():
            plsc.subcore_barrier()
            plsc.subcore_barrier()

    f = pl.kernel(
        body,
        out_type=jax.ShapeDtypeStruct((npad,), jnp.float32),
        mesh=_sc_mesh(),
        scratch_types=[
            pltpu.VMEM((nchunks, CHUNK), jnp.int32),
            pltpu.VMEM((CHUNK,), jnp.float32),
            pltpu.VMEM((rpt,), jnp.float32),
            pltpu.VMEM_SHARED((npad,), jnp.float32),
            pltpu.SemaphoreType.DMA,
        ],
    )
    return f(dstT)


def _gat_edge_sc(h2, asd, dpack, srcT64, dstT64, npad):
    CH = 16
    nchunks = srcT64.shape[0] // NS
    rpt = npad // NS

    def body(h_ref, asd_ref, dp_ref, src_hbm, dst_hbm, num_ref, den_ref,
             sidx, didx, tidx, gbuf, abuf, dbuf, exbuf, widx, acc, dacc,
             dstage,
             semi0, semi1, semd0, semd1, semg0, semg1, sema0, sema1,
             semp0, semp1):
        cc = lax.axis_index("c")
        ss = lax.axis_index("s")
        semi = (semi0, semi1)
        semd = (semd0, semd1)
        semg = (semg0, semg1)
        sema = (sema0, sema1)
        semp = (semp0, semp1)
        iota = lax.iota(jnp.int32, LANES)
        zero = jnp.zeros((LANES,), jnp.float32)

        def zg(i, _):
            gbuf[0, i // 8, pl.ds((i % 8) * LANES, LANES)] = zero
            return 0
        lax.fori_loop(0, CH * 8, zg, 0)
        def ze(i, _):
            exbuf[0, i, pl.ds(0, LANES)] = zero
            return 0
        lax.fori_loop(0, CH, ze, 0)
        def zacc(p, _):
            pltpu.sync_copy(
                gbuf.at[0], acc.at[pl.ds(ss * rpt + p * CH, CH)])
            pltpu.sync_copy(
                exbuf.at[0], dacc.at[pl.ds(ss * rpt + p * CH, CH)])
            return 0
        lax.fori_loop(0, rpt // CH, zacc, 0)

        def idx_dma(j, b):
            pltpu.make_async_copy(
                src_hbm.at[ss * nchunks + j], sidx.at[b], semi[b]).start()
            pltpu.make_async_copy(
                dst_hbm.at[ss * nchunks + j], didx.at[b], semd[b]).start()

        def stage_gathers(j, b):
            pltpu.make_async_copy(
                src_hbm.at[ss * nchunks + j], sidx.at[b], semi[b]).wait()
            pltpu.make_async_copy(
                dst_hbm.at[ss * nchunks + j], didx.at[b], semd[b]).wait()
            def tr(k, _):
                tidx[b, pl.ds(k * LANES, LANES)] = (
                    sidx[b, pl.ds(k * LANES, LANES)] * 2 + cc)
                return 0
            lax.fori_loop(0, CH // LANES, tr, 0)
            pltpu.make_async_copy(
                h_ref.at[tidx.at[b]], gbuf.at[b], semg[b]).start()
            pltpu.make_async_copy(
                asd_ref.at[sidx.at[b]], abuf.at[b], sema[b]).start()
            pltpu.make_async_copy(
                dp_ref.at[didx.at[b]], dbuf.at[b], semp[b]).start()
            pltpu.make_async_copy(
                asd_ref.at[sidx.at[b]], abuf.at[b], sema[b]).wait()
            pltpu.make_async_copy(
                dp_ref.at[didx.at[b]], dbuf.at[b], semp[b]).wait()

        def compute_scatter(j, b):
            def ex_loop(e, _):
                asv = abuf[b, e, pl.ds(0, LANES)]
                adv = dbuf[b, e, pl.ds(0, LANES)]
                mv = dbuf[b, e, pl.ds(LANES, LANES)]
                al = asv + adv
                al = jnp.where(al >= 0.0, al, al * 0.2)
                exbuf[b, e, pl.ds(0, LANES)] = jnp.exp(al - mv)
                return 0
            lax.fori_loop(0, CH, ex_loop, 0)

            pltpu.make_async_copy(
                h_ref.at[tidx.at[b]], gbuf.at[b], semg[b]).wait()
            def sc_loop(e, _):
                exv = exbuf[b, e, pl.ds(0, LANES)]
                for hh in range(4):
                    col = cc * 4 + hh
                    bc = lax.gather(
                        exv,
                        jnp.full((LANES, 1), col, jnp.int32),
                        lax.GatherDimensionNumbers(
                            offset_dims=(), collapsed_slice_dims=(0,),
                            start_index_map=(0,)),
                        (1,),
                        mode=lax.GatherScatterMode.PROMISE_IN_BOUNDS)
                    gbuf[b, e, pl.ds(hh * 32, LANES)] = (
                        gbuf[b, e, pl.ds(hh * 32, LANES)] * bc)
                    gbuf[b, e, pl.ds(hh * 32 + LANES, LANES)] = (
                        gbuf[b, e, pl.ds(hh * 32 + LANES, LANES)] * bc)
                return 0
            lax.fori_loop(0, CH, sc_loop, 0)

            pltpu.sync_copy(gbuf.at[b], acc.at[didx.at[b]], add=True)
            @pl.when(cc == 0)
            def _():
                pltpu.sync_copy(exbuf.at[b], dacc.at[didx.at[b]], add=True)

        plsc.subcore_barrier()

        idx_dma(0, 0)
        stage_gathers(0, 0)
        if nchunks > 1:
            idx_dma(1, 1)

        def step(jj, _):
            j = jj * 2
            for b in range(2):
                jb = j + b
                nb = 1 - b
                @pl.when(jb + 1 < nchunks)
                def _():
                    stage_gathers(jb + 1, nb)
                @pl.when(jb < nchunks)
                def _():
                    compute_scatter(jb, b)
                @pl.when(jb + 2 < nchunks)
                def _():
                    idx_dma(jb + 2, b)
            return 0
        lax.fori_loop(0, (nchunks + 1) // 2, step, 0)

        plsc.subcore_barrier()

        def wb(p, _):
            def mkidx(k, _):
                widx[0, pl.ds(k * LANES, LANES)] = (
                    iota + (ss * rpt + p * CH + k * LANES)) * 2 + cc
                return 0
            lax.fori_loop(0, CH // LANES, mkidx, 0)
            pltpu.sync_copy(
                acc.at[pl.ds(ss * rpt + p * CH, CH)], gbuf.at[0])
            pltpu.sync_copy(gbuf.at[0], num_ref.at[widx.at[0]])
            return 0
        lax.fori_loop(0, rpt // CH, wb, 0)

        @pl.when(cc == 0)
        def _():
            def dwb(q, _):
                pltpu.sync_copy(
                    dacc.at[pl.ds(ss * rpt + q * 128, 128)], dstage)
                def pk(r, _):
                    gbuf[0, r // 8, pl.ds((r % 8) * LANES, LANES)] = (
                        dstage[r, pl.ds(0, LANES)])
                    return 0
                lax.fori_loop(0, 128, pk, 0)
                pltpu.sync_copy(
                    gbuf.at[0, pl.ds(0, LANES)],
                    den_ref.at[pl.ds(
                        pl.multiple_of((ss * rpt + q * 128) // 8, 8),
                        LANES)])
                return 0
            lax.fori_loop(0, rpt // 128, dwb, 0)


    f = pl.kernel(
        body,
        out_type=(jax.ShapeDtypeStruct((2 * npad, 128), jnp.float32),
                  jax.ShapeDtypeStruct((npad // 8, 128), jnp.float32)),
        mesh=_sc_mesh(),
        scratch_types=[
            pltpu.VMEM((2, CH), jnp.int32),
            pltpu.VMEM((2, CH), jnp.int32),
            pltpu.VMEM((2, CH), jnp.int32),
            pltpu.VMEM((2, CH, 128), jnp.float32),
            pltpu.VMEM((2, CH, 128), jnp.float32),
            pltpu.VMEM((2, CH, 128), jnp.float32),
            pltpu.VMEM((2, CH, 16), jnp.float32),
            pltpu.VMEM((1, CH), jnp.int32),
            pltpu.VMEM_SHARED((npad, 128), jnp.float32),
            pltpu.VMEM_SHARED((npad, 16), jnp.float32),
            pltpu.VMEM((128, 16), jnp.float32),
            pltpu.SemaphoreType.DMA,
            pltpu.SemaphoreType.DMA,
            pltpu.SemaphoreType.DMA,
            pltpu.SemaphoreType.DMA,
            pltpu.SemaphoreType.DMA,
            pltpu.SemaphoreType.DMA,
            pltpu.SemaphoreType.DMA,
            pltpu.SemaphoreType.DMA,
            pltpu.SemaphoreType.DMA,
            pltpu.SemaphoreType.DMA,
        ],
    )
    return f(h2, asd, dpack, srcT64, dstT64)


def _dense_gat_body(x_ref, w_ref, asd_w_ref, h_ref, asd_ref):
    h = jnp.dot(x_ref[...], w_ref[...], preferred_element_type=jnp.float32)
    h_ref[...] = h
    asd_ref[...] = jnp.dot(h, asd_w_ref[...],
                           preferred_element_type=jnp.float32)


def _dense_gat(x, W, asd_w, block_m=1024):
    M, K = x.shape
    N = W.shape[1]
    return pl.pallas_call(
        _dense_gat_body,
        grid=(pl.cdiv(M, block_m),),
        in_specs=[
            pl.BlockSpec((block_m, K), lambda i: (i, 0)),
            pl.BlockSpec((K, N), lambda i: (0, 0)),
            pl.BlockSpec((N, 16), lambda i: (0, 0)),
        ],
        out_specs=(pl.BlockSpec((block_m, N), lambda i: (i, 0)),
                   pl.BlockSpec((block_m, 16), lambda i: (i, 0))),
        out_shape=(jax.ShapeDtypeStruct((M, N), jnp.float32),
                   jax.ShapeDtypeStruct((M, 16), jnp.float32)),
    )(x, W, asd_w)


def _amax_body(asd_ref, o_ref):
    o_ref[...] = jnp.max(asd_ref[..., 0:8], keepdims=True)[0:1, 0:1]


def _amax(asd):
    M = asd.shape[0]
    return pl.pallas_call(
        _amax_body,
        out_shape=jax.ShapeDtypeStruct((1, 1), jnp.float32),
    )(asd)


def _gat_pack_body(asd_ref, a_ref, s_ref, d_ref):
    bm = asd_ref.shape[0]
    a_s = asd_ref[..., 0:8]
    ad = asd_ref[..., 8:16]
    am = a_ref[0, 0] + ad
    m = jnp.where(am >= 0.0, am, am * 0.2)
    z8 = jnp.zeros((bm, 8), jnp.float32)
    z104 = jnp.zeros((bm, 104), jnp.float32)
    z120 = jnp.zeros((bm, 120), jnp.float32)
    s_ref[...] = jnp.concatenate([a_s, z120], axis=-1)
    d_ref[...] = jnp.concatenate([ad, z8, m, z104], axis=-1)


def _gat_pack(asd, A, block_m=1024):
    M = asd.shape[0]
    return pl.pallas_call(
        _gat_pack_body,
        grid=(pl.cdiv(M, block_m),),
        in_specs=[
            pl.BlockSpec((block_m, 16), lambda i: (i, 0)),
            pl.BlockSpec((1, 1), lambda i: (0, 0)),
        ],
        out_specs=(pl.BlockSpec((block_m, 128), lambda i: (i, 0)),
                   pl.BlockSpec((block_m, 128), lambda i: (i, 0))),
        out_shape=(jax.ShapeDtypeStruct((M, 128), jnp.float32),
                   jax.ShapeDtypeStruct((M, 128), jnp.float32)),
    )(asd, A)


def _gat_finish_body(num_ref, den_ref, asd_ref, a_ref, h_ref, b_ref, o_ref):
    bm = num_ref.shape[0]
    a_s = asd_ref[..., 0:8]
    a_d = asd_ref[..., 8:16]
    am = a_ref[0, 0] + a_d
    m = jnp.where(am >= 0.0, am, am * 0.2)
    al = a_s + a_d
    al = jnp.where(al >= 0.0, al, al * 0.2)
    ex_self = jnp.exp(al - m)
    den = den_ref[..., 0:8] + ex_self
    exb = jnp.broadcast_to(ex_self[:, :, None], (bm, 8, 32)).reshape(bm, 256)
    denb = jnp.broadcast_to(
        (den + 1e-16)[:, :, None], (bm, 8, 32)).reshape(bm, 256)
    out = (num_ref[...] + exb * h_ref[...]) / denb
    o_ref[...] = jnp.maximum(out + b_ref[...], 0.0)


def _gat_finish(num, den, asd, A, h, b, block_m=1024):
    M = num.shape[0]
    return pl.pallas_call(
        _gat_finish_body,
        grid=(pl.cdiv(M, block_m),),
        in_specs=[
            pl.BlockSpec((block_m, 256), lambda i: (i, 0)),
            pl.BlockSpec((block_m, 16), lambda i: (i, 0)),
            pl.BlockSpec((block_m, 16), lambda i: (i, 0)),
            pl.BlockSpec((1, 1), lambda i: (0, 0)),
            pl.BlockSpec((block_m, 256), lambda i: (i, 0)),
            pl.BlockSpec((1, 256), lambda i: (0, 0)),
        ],
        out_specs=pl.BlockSpec((block_m, 256), lambda i: (i, 0)),
        out_shape=jax.ShapeDtypeStruct((M, 256), jnp.float32),
    )(num, den, asd, A, h, b.reshape(1, 256))


def _gat_layer_xla(a, src, dst, lp, n):
    h = _dense(a, lp["W"], lp["b"] * 0.0).reshape(n, HEADS, HCH)
    a_s = jnp.sum(h * lp["att_s"][None], axis=-1)
    a_d = jnp.sum(h * lp["att_d"][None], axis=-1)
    alpha = jax.nn.leaky_relu(a_s[src] + a_d[dst], 0.2)
    amax = jax.ops.segment_max(alpha, dst, num_segments=n)
    amax = jnp.where(jnp.isfinite(amax), amax, 0.0)
    ex = jnp.exp(alpha - amax[dst])
    den = jax.ops.segment_sum(ex, dst, num_segments=n)
    num = jax.ops.segment_sum(h[src] * ex[:, :, None], dst, num_segments=n)
    out = num / (den[:, :, None] + 1e-16)
    return jnp.maximum(out.reshape(n, HEADS * HCH) + lp["b"], 0.0)


def kernel(x, edge_index, params):
    n = x.shape[0]
    src0, dst0 = edge_index[0], edge_index[1]
    E = src0.shape[0]

    npad = ((n + 16 + NS * 128 - 1) // (NS * 128)) * (NS * 128)
    ept = ((E + NS * CHUNK - 1) // (NS * CHUNK)) * CHUNK
    epad = ept * NS
    pad = epad - E
    pidx = jnp.arange(pad, dtype=src0.dtype)
    src_p = jnp.concatenate([src0, pidx % n])
    dst_p = jnp.concatenate([dst0, n + (pidx % 16)])
    srcT = src_p.reshape(NS, ept // CHUNK, CHUNK)
    dstT = dst_p.reshape(NS, ept // CHUNK, CHUNK)
    srcT2 = src_p.reshape(NS * (ept // CHUNK), CHUNK)
    dstT2 = dst_p.reshape(NS * (ept // CHUNK), CHUNK)

    x_pad = jnp.pad(x, ((0, npad - n), (0, 0)))

    cnt = _edge_count_sc(dstT, npad)
    dis, inv_cnt = _degs(cnt)

    def agg256(h):
        out2 = _seg_sum_sc(h.reshape(2 * npad, 128), srcT2, dstT2, npad)
        return out2.reshape(npad, 256)

    g = x_pad
    for lp in params["gcn"]:
        hp = _dense_rowscale(g, lp["W"], dis)
        agg = agg256(hp)
        g = _gcn_finish(agg, hp, dis, lp["b"])

    s_ = x_pad
    for lp in params["sage"]:
        hl = _dense(s_, lp["Wl"], lp["bl"] * 0.0)
        hr = _dense(s_, lp["Wr"], lp["bl"] * 0.0)
        agg = agg256(hl)
        s_ = _sage_finish(agg, hr, inv_cnt, lp["bl"])

    srcT32 = src_p.reshape(NS * (ept // 16), 16)
    dstT32 = dst_p.reshape(NS * (ept // 16), 16)
    eye8 = jnp.eye(8, dtype=jnp.float32)
    loopx = jnp.arange(n, dtype=src0.dtype)
    srcl = jnp.concatenate([src0, loopx])
    dstl = jnp.concatenate([dst0, loopx])
    a_x = x
    for lp in params["gat"]:
        a_x = _gat_layer_xla(a_x, srcl, dstl, lp, n)
    a = jnp.pad(a_x, ((0, npad - n), (0, 0)))
    for lp in []:
        bs = (lp["att_s"][:, :, None] * eye8[:, None, :]).reshape(256, 8)
        bd = (lp["att_d"][:, :, None] * eye8[:, None, :]).reshape(256, 8)
        asd_w = jnp.concatenate([bs, bd], axis=1)
        h, asd = _dense_gat(a, lp["W"], asd_w)
        A = _amax(asd)
        sp128, dp128 = _gat_pack(asd, A)
        num2, den128 = _gat_edge_sc(
            h.reshape(2 * npad, 128), sp128, dp128, srcT32, dstT32, npad)
        a = _gat_finish(num2.reshape(npad, 256), den128.reshape(npad, 16),
                        asd, A, h, lp["b"])

    f = jnp.concatenate([g[:n], a[:n], s_[:n]], axis=1)
    f = _dense(f, params["f1W"], params["f1b"], act="relu")
    f = _dense(f, params["f2W"], params["f2b"], act="relu")
    warp = _dense(f, params["wW"], params["wb"])
    v = _dense(f, params["v1W"], params["v1b"], act="relu")
    v = _dense(v, params["v2W"], params["v2b"], act="sigmoid")
    return warp, v
